# Initial kernel scaffold; baseline (speedup 1.0000x reference)
import functools

import jax
import jax.numpy as jnp
from jax import lax
from jax.experimental import pallas as pl
from jax.experimental.pallas import tpu as pltpu
from jax.experimental.pallas import tpu_sc as plsc

NUM_FIELDS = 26
VOCAB = 100000
EMBED_DIM = 32
BATCH = 16384

TOTAL_ROWS = BATCH * NUM_FIELDS
NUM_CORES = 2
NUM_SUBCORES = 16
NW = NUM_CORES * NUM_SUBCORES
PER_W = TOTAL_ROWS // NW
CHUNK = 128
NCHUNK = PER_W // CHUNK


def _make_kernel(interpret=False):
    mesh = plsc.VectorSubcoreMesh(
        core_axis_name="c", subcore_axis_name="s",
        num_cores=NUM_CORES, num_subcores=NUM_SUBCORES)

    @functools.partial(
        pl.kernel,
        mesh=mesh,
        out_type=jax.ShapeDtypeStruct((TOTAL_ROWS, EMBED_DIM), jnp.float32),
        scratch_types=[
            pltpu.VMEM((NCHUNK, CHUNK), jnp.int32),
            pltpu.VMEM((CHUNK, EMBED_DIM), jnp.float32),
            pltpu.SemaphoreType.DMA,
        ],
        interpret=interpret,
    )
    def gather_kernel(tab_hbm, idx_hbm, out_hbm, idx_v, buf, gsem):
        wid = lax.axis_index("s") * NUM_CORES + lax.axis_index("c")
        base = wid * PER_W
        pltpu.sync_copy(idx_hbm.at[wid], idx_v)

        def body(j, carry):
            pltpu.async_copy(tab_hbm.at[idx_v.at[j]], buf, gsem).wait()
            pltpu.sync_copy(buf, out_hbm.at[pl.ds(base + j * CHUNK, CHUNK)])
            return carry

        lax.fori_loop(0, NCHUNK, body, 0)

    return gather_kernel


_gather = _make_kernel()


@jax.jit
def kernel(x, tables):
    flat_tables = tables.reshape(NUM_FIELDS * VOCAB, EMBED_DIM)
    offsets = jnp.arange(NUM_FIELDS, dtype=jnp.int32) * VOCAB
    flat_idx = (x.astype(jnp.int32) + offsets[None, :]).reshape(
        NW, NCHUNK, CHUNK)
    out = _gather(flat_tables, flat_idx)
    return out.reshape(BATCH, NUM_FIELDS, EMBED_DIM)


# speedup vs baseline: 1.0962x; 1.0962x over previous
import functools

import jax
import jax.numpy as jnp
from jax import lax
from jax.experimental import pallas as pl
from jax.experimental.pallas import tpu as pltpu
from jax.experimental.pallas import tpu_sc as plsc

NUM_FIELDS = 26
VOCAB = 100000
EMBED_DIM = 32
BATCH = 16384

TOTAL_ROWS = BATCH * NUM_FIELDS
NUM_CORES = 2
NUM_SUBCORES = 16
NW = NUM_CORES * NUM_SUBCORES
PER_W = TOTAL_ROWS // NW
CHUNK = 128
NCHUNK = PER_W // CHUNK


def _make_kernel(interpret=False):
    mesh = plsc.VectorSubcoreMesh(
        core_axis_name="c", subcore_axis_name="s",
        num_cores=NUM_CORES, num_subcores=NUM_SUBCORES)

    @functools.partial(
        pl.kernel,
        mesh=mesh,
        out_type=jax.ShapeDtypeStruct((TOTAL_ROWS, EMBED_DIM), jnp.float32),
        scratch_types=[
            pltpu.VMEM((NCHUNK, CHUNK), jnp.int32),
            pltpu.VMEM((CHUNK, EMBED_DIM), jnp.float32),
            pltpu.SemaphoreType.DMA,
        ],
        compiler_params=pltpu.CompilerParams(use_tc_tiling_on_sc=False),
        interpret=interpret,
    )
    def gather_kernel(tab_hbm, idx_hbm, out_hbm, idx_v, buf, gsem):
        wid = lax.axis_index("s") * NUM_CORES + lax.axis_index("c")
        base = wid * PER_W
        pltpu.sync_copy(idx_hbm.at[wid], idx_v)

        def body(j, carry):
            pltpu.async_copy(tab_hbm.at[idx_v.at[j]], buf, gsem).wait()
            pltpu.sync_copy(buf, out_hbm.at[pl.ds(base + j * CHUNK, CHUNK)])
            return carry

        lax.fori_loop(0, NCHUNK, body, 0)

    return gather_kernel


_gather = _make_kernel()


@jax.jit
def kernel(x, tables):
    flat_tables = tables.reshape(NUM_FIELDS * VOCAB, EMBED_DIM)
    offsets = jnp.arange(NUM_FIELDS, dtype=jnp.int32) * VOCAB
    flat_idx = (x.astype(jnp.int32) + offsets[None, :]).reshape(
        NW, NCHUNK, CHUNK)
    out = _gather(flat_tables, flat_idx)
    return out.reshape(BATCH, NUM_FIELDS, EMBED_DIM)


# speedup vs baseline: 1.1501x; 1.0492x over previous
import functools

import jax
import jax.numpy as jnp
from jax import lax
from jax.experimental import pallas as pl
from jax.experimental.pallas import tpu as pltpu
from jax.experimental.pallas import tpu_sc as plsc

NUM_FIELDS = 26
VOCAB = 100000
EMBED_DIM = 32
BATCH = 16384

TOTAL_ROWS = BATCH * NUM_FIELDS
NUM_CORES = 2
NUM_SUBCORES = 16
NW = NUM_CORES * NUM_SUBCORES
PER_W = TOTAL_ROWS // NW
CHUNK = 128
NCHUNK = PER_W // CHUNK
G = 13
GROUP_ROWS = G * CHUNK
NGROUP = NCHUNK // G
NBUF = 2


def _make_kernel(interpret=False):
    mesh = plsc.VectorSubcoreMesh(
        core_axis_name="c", subcore_axis_name="s",
        num_cores=NUM_CORES, num_subcores=NUM_SUBCORES)

    @functools.partial(
        pl.kernel,
        mesh=mesh,
        out_type=jax.ShapeDtypeStruct((TOTAL_ROWS, EMBED_DIM), jnp.float32),
        scratch_types=[
            pltpu.VMEM((NCHUNK, CHUNK), jnp.int32),
            pltpu.VMEM((NBUF, GROUP_ROWS, EMBED_DIM), jnp.float32),
            pltpu.SemaphoreType.DMA,
            pltpu.SemaphoreType.DMA,
            pltpu.SemaphoreType.DMA,
        ],
        compiler_params=pltpu.CompilerParams(use_tc_tiling_on_sc=False),
        interpret=interpret,
    )
    def gather_kernel(tab_hbm, idx_hbm, out_hbm, idx_v, bufs, gsem, wsem0,
                      wsem1):
        wid = lax.axis_index("s") * NUM_CORES + lax.axis_index("c")
        base = wid * PER_W
        pltpu.sync_copy(idx_hbm.at[wid], idx_v)
        wsems = (wsem0, wsem1)

        @pl.loop(0, NGROUP, step=NBUF)
        def outer(g0):
            for b in range(NBUF):
                g = g0 + b
                buf = bufs.at[b]
                out_slice = out_hbm.at[pl.ds(base + g * GROUP_ROWS,
                                             GROUP_ROWS)]

                @pl.when(g >= NBUF)
                def _():
                    pltpu.make_async_copy(buf, out_slice, wsems[b]).wait()

                descs = [
                    pltpu.async_copy(
                        tab_hbm.at[idx_v.at[g * G + k]],
                        buf.at[pl.ds(k * CHUNK, CHUNK)], gsem)
                    for k in range(G)
                ]
                for d in descs:
                    d.wait()
                pltpu.async_copy(buf, out_slice, wsems[b])

        for b in range(NBUF):
            pltpu.make_async_copy(
                bufs.at[b], out_hbm.at[pl.ds(base, GROUP_ROWS)],
                wsems[b]).wait()

    return gather_kernel


_gather = _make_kernel()


@jax.jit
def kernel(x, tables):
    flat_tables = tables.reshape(NUM_FIELDS * VOCAB, EMBED_DIM)
    offsets = jnp.arange(NUM_FIELDS, dtype=jnp.int32) * VOCAB
    flat_idx = (x.astype(jnp.int32) + offsets[None, :]).reshape(
        NW, NCHUNK, CHUNK)
    out = _gather(flat_tables, flat_idx)
    return out.reshape(BATCH, NUM_FIELDS, EMBED_DIM)


# speedup vs baseline: 3.8272x; 3.3277x over previous
import functools

import jax
import jax.numpy as jnp
from jax import lax
from jax.experimental import pallas as pl
from jax.experimental.pallas import tpu as pltpu
from jax.experimental.pallas import tpu_sc as plsc

NUM_FIELDS = 26
VOCAB = 100000
EMBED_DIM = 32
BATCH = 16384

NUM_CORES = 2
NUM_SUBCORES = 16
NW = NUM_CORES * NUM_SUBCORES
LANES = 16
HALF = BATCH // 2


def _make_kernel(interpret=False):
    mesh = plsc.VectorSubcoreMesh(
        core_axis_name="c", subcore_axis_name="s",
        num_cores=NUM_CORES, num_subcores=NUM_SUBCORES)

    @functools.partial(
        pl.kernel,
        mesh=mesh,
        out_type=jax.ShapeDtypeStruct((NUM_FIELDS, EMBED_DIM, BATCH),
                                      jnp.float32),
        scratch_types=[
            pltpu.VMEM((VOCAB,), jnp.float32),
            pltpu.VMEM((BATCH,), jnp.int32),
            pltpu.VMEM((HALF,), jnp.float32),
        ],
        compiler_params=pltpu.CompilerParams(use_tc_tiling_on_sc=True,
                                             needs_layout_passes=False),
        interpret=interpret,
    )
    def gather_kernel(tab_hbm, idx_hbm, out_hbm, tv, xv, ov):
        wid = lax.axis_index("s") * NUM_CORES + lax.axis_index("c")

        def per_field(f, carry):
            pltpu.sync_copy(idx_hbm.at[f], xv)
            pltpu.sync_copy(tab_hbm.at[f, wid], tv)
            for h in range(2):

                @pl.loop(0, HALF // LANES, unroll=8)
                def inner(i):
                    idx = xv[pl.ds(h * HALF + i * LANES, LANES)]
                    ov[pl.ds(i * LANES, LANES)] = plsc.load_gather(
                        tv, [idx])

                pltpu.sync_copy(ov, out_hbm.at[f, wid,
                                               pl.ds(h * HALF, HALF)])
            return carry

        lax.fori_loop(0, NUM_FIELDS, per_field, 0)

    return gather_kernel


_gather = _make_kernel()


@jax.jit
def kernel(x, tables):
    tables_t = jnp.transpose(tables, (0, 2, 1))
    x_t = jnp.transpose(x.astype(jnp.int32), (1, 0))
    out_t = _gather(tables_t, x_t)
    return jnp.transpose(out_t, (2, 0, 1))
